# Initial kernel scaffold; baseline (speedup 1.0000x reference)
import jax
import jax.numpy as jnp
from jax.experimental import pallas as pl


def kernel(x, edge_index):
    raise NotImplementedError("write your pallas kernel here")


# speedup vs baseline: 2.3405x; 2.3405x over previous
import functools

import jax
import jax.numpy as jnp
from jax import lax
from jax.experimental import pallas as pl
from jax.experimental.pallas import tpu as pltpu, tpu_sc as plsc

N_NODES = 10000
N_EDGES = 320000
D_FEAT = 128
NUM_ITERS = 3

L = 16
NC, NS = 2, 16
NW = NC * NS
F = D_FEAT // NW

CHUNK = 2048
EPAD = ((N_EDGES + CHUNK - 1) // CHUNK) * CHUNK
NCHUNK = EPAD // CHUNK
STEPS = CHUNK // L

NROW = N_NODES + L
XSZ = F * N_NODES
ASZ = F * NROW


def _wl_body(xt_hbm, src_hbm, dst_hbm, out_hbm,
             x_v, agg_v, inv_v, src_v, dst_v):
    wid = lax.axis_index("s") * NC + lax.axis_index("c")

    pltpu.sync_copy(xt_hbm.at[pl.ds(wid * XSZ, XSZ)], x_v)
    pltpu.sync_copy(x_v, out_hbm.at[pl.ds(wid * XSZ, XSZ)])

    zeros = jnp.zeros((L,), jnp.float32)
    ones = jnp.ones((L,), jnp.float32)
    half = jnp.float32(0.5)

    for it in range(1, NUM_ITERS + 1):
        def zero_body(j, _):
            agg_v[pl.ds(j * L, L)] = zeros
            return 0
        lax.fori_loop(0, ASZ // L, zero_body, 0)
        if it == 1:
            def zero_deg(j, _):
                inv_v[pl.ds(j * L, L)] = zeros
                return 0
            lax.fori_loop(0, NROW // L, zero_deg, 0)

        def chunk_body(c, _):
            pltpu.sync_copy(src_hbm.at[pl.ds(c * CHUNK, CHUNK)], src_v)
            pltpu.sync_copy(dst_hbm.at[pl.ds(c * CHUNK, CHUNK)], dst_v)

            def step(i, _):
                s16 = src_v[pl.ds(i * L, L)]
                d16 = dst_v[pl.ds(i * L, L)]
                for f in range(F):
                    g = plsc.load_gather(x_v, [s16 + f * N_NODES])
                    plsc.addupdate_scatter(agg_v, [d16 + f * NROW], g)
                if it == 1:
                    plsc.addupdate_scatter(inv_v, [d16], ones)
                return 0
            lax.fori_loop(0, STEPS, step, 0)
            return 0
        lax.fori_loop(0, NCHUNK, chunk_body, 0)

        if it == 1:
            def inv_body(j, _):
                d = inv_v[pl.ds(j * L, L)]
                inv_v[pl.ds(j * L, L)] = jnp.where(
                    d > 0.0, 1.0 / jnp.maximum(d, 1.0), 0.0)
                return 0
            lax.fori_loop(0, NROW // L, inv_body, 0)

        def upd_body(j, _):
            iv = inv_v[pl.ds(j * L, L)]
            for f in range(F):
                xo = x_v[pl.ds(f * N_NODES + j * L, L)]
                ag = agg_v[pl.ds(f * NROW + j * L, L)]
                x_v[pl.ds(f * N_NODES + j * L, L)] = half * (xo + ag * iv)
            return 0
        lax.fori_loop(0, N_NODES // L, upd_body, 0)

        pltpu.sync_copy(
            x_v, out_hbm.at[pl.ds((it * NW + wid) * XSZ, XSZ)])


@functools.partial(jax.jit, static_argnums=())
def _wl_sc(xt_flat, src, dst):
    mesh = plsc.VectorSubcoreMesh(core_axis_name="c", subcore_axis_name="s")
    return pl.kernel(
        _wl_body,
        out_type=jax.ShapeDtypeStruct(((NUM_ITERS + 1) * D_FEAT * N_NODES,),
                                      jnp.float32),
        mesh=mesh,
        scratch_types=[
            pltpu.VMEM((XSZ,), jnp.float32),
            pltpu.VMEM((ASZ,), jnp.float32),
            pltpu.VMEM((NROW,), jnp.float32),
            pltpu.VMEM((CHUNK,), jnp.int32),
            pltpu.VMEM((CHUNK,), jnp.int32),
        ],
        compiler_params=pltpu.CompilerParams(needs_layout_passes=False),
    )(xt_flat, src, dst)


def kernel(x, edge_index):
    xt = x.T.reshape(-1)
    pad = EPAD - N_EDGES
    src = jnp.concatenate([edge_index[0], jnp.zeros((pad,), jnp.int32)])
    dst = jnp.concatenate(
        [edge_index[1], jnp.full((pad,), N_NODES, jnp.int32)])
    out_t = _wl_sc(xt, src, dst)
    return out_t.reshape((NUM_ITERS + 1) * D_FEAT, N_NODES).T


# speedup vs baseline: 3.0770x; 1.3147x over previous
import functools

import jax
import jax.numpy as jnp
from jax import lax
from jax.experimental import pallas as pl
from jax.experimental.pallas import tpu as pltpu, tpu_sc as plsc

N_NODES = 10000
N_EDGES = 320000
D_FEAT = 128
NUM_ITERS = 3

L = 16
NC, NS = 2, 16
NW = NC * NS
F = D_FEAT // NW

CHUNK = 8192
EPAD = ((N_EDGES + CHUNK - 1) // CHUNK) * CHUNK
NCHUNK = EPAD // CHUNK
NPAIR = NCHUNK // 2
STEPS = CHUNK // L
UNROLL = 8

NROW = N_NODES + L
XSZ = F * N_NODES
ASZ = F * NROW


def _wl_body(xt_hbm, src_hbm, dst_hbm, out_hbm,
             x_v, agg_v, inv_v, sA, dA, sB, dB,
             sem_sa, sem_da, sem_sb, sem_db):
    wid = lax.axis_index("s") * NC + lax.axis_index("c")

    pltpu.sync_copy(xt_hbm.at[pl.ds(wid * XSZ, XSZ)], x_v)
    pltpu.sync_copy(x_v, out_hbm.at[pl.ds(wid * XSZ, XSZ)])

    zeros = jnp.zeros((L,), jnp.float32)
    ones = jnp.ones((L,), jnp.float32)
    half = jnp.float32(0.5)

    def start(c, buf_s, buf_d, ss, sd):
        pltpu.make_async_copy(
            src_hbm.at[pl.ds(c * CHUNK, CHUNK)], buf_s, ss).start()
        pltpu.make_async_copy(
            dst_hbm.at[pl.ds(c * CHUNK, CHUNK)], buf_d, sd).start()

    def wait(buf_s, buf_d, ss, sd):
        pltpu.make_async_copy(
            src_hbm.at[pl.ds(0, CHUNK)], buf_s, ss).wait()
        pltpu.make_async_copy(
            dst_hbm.at[pl.ds(0, CHUNK)], buf_d, sd).wait()

    for it in range(1, NUM_ITERS + 1):
        do_deg = it == 1

        def zero_body(j, _):
            for u in range(UNROLL):
                agg_v[pl.ds(j * (UNROLL * L) + u * L, L)] = zeros
            return 0
        lax.fori_loop(0, ASZ // (UNROLL * L), zero_body, 0)
        if do_deg:
            def zero_deg(j, _):
                inv_v[pl.ds(j * L, L)] = zeros
                return 0
            lax.fori_loop(0, NROW // L, zero_deg, 0)

        def process(buf_s, buf_d):
            def step(i, _):
                for u in range(UNROLL):
                    off = i * (UNROLL * L) + u * L
                    s16 = buf_s[pl.ds(off, L)]
                    d16 = buf_d[pl.ds(off, L)]
                    for f in range(F):
                        g = plsc.load_gather(x_v, [s16 + f * N_NODES])
                        plsc.addupdate_scatter(
                            agg_v, [d16 + f * NROW], g)
                    if do_deg:
                        plsc.addupdate_scatter(inv_v, [d16], ones)
                return 0
            lax.fori_loop(0, STEPS // UNROLL, step, 0)

        start(0, sA, dA, sem_sa, sem_da)
        start(1, sB, dB, sem_sb, sem_db)

        def pair_body(t, _):
            wait(sA, dA, sem_sa, sem_da)
            process(sA, dA)
            start(2 * t + 2, sA, dA, sem_sa, sem_da)
            wait(sB, dB, sem_sb, sem_db)
            process(sB, dB)
            start(2 * t + 3, sB, dB, sem_sb, sem_db)
            return 0
        lax.fori_loop(0, NPAIR - 1, pair_body, 0)
        wait(sA, dA, sem_sa, sem_da)
        process(sA, dA)
        wait(sB, dB, sem_sb, sem_db)
        process(sB, dB)

        if do_deg:
            def inv_body(j, _):
                d = inv_v[pl.ds(j * L, L)]
                inv_v[pl.ds(j * L, L)] = jnp.where(
                    d > 0.0, 1.0 / jnp.maximum(d, 1.0), 0.0)
                return 0
            lax.fori_loop(0, NROW // L, inv_body, 0)

        def upd_body(j, _):
            base = j * (5 * L)
            for u in range(5):
                iv = inv_v[pl.ds(base + u * L, L)]
                for f in range(F):
                    xo = x_v[pl.ds(f * N_NODES + base + u * L, L)]
                    ag = agg_v[pl.ds(f * NROW + base + u * L, L)]
                    x_v[pl.ds(f * N_NODES + base + u * L, L)] = (
                        half * (xo + ag * iv))
            return 0
        lax.fori_loop(0, N_NODES // (5 * L), upd_body, 0)

        pltpu.sync_copy(
            x_v, out_hbm.at[pl.ds((it * NW + wid) * XSZ, XSZ)])


@functools.partial(jax.jit, static_argnums=())
def _wl_sc(xt_flat, src, dst):
    mesh = plsc.VectorSubcoreMesh(core_axis_name="c", subcore_axis_name="s")
    return pl.kernel(
        _wl_body,
        out_type=jax.ShapeDtypeStruct(((NUM_ITERS + 1) * D_FEAT * N_NODES,),
                                      jnp.float32),
        mesh=mesh,
        scratch_types=[
            pltpu.VMEM((XSZ,), jnp.float32),
            pltpu.VMEM((ASZ,), jnp.float32),
            pltpu.VMEM((NROW,), jnp.float32),
            pltpu.VMEM((CHUNK,), jnp.int32),
            pltpu.VMEM((CHUNK,), jnp.int32),
            pltpu.VMEM((CHUNK,), jnp.int32),
            pltpu.VMEM((CHUNK,), jnp.int32),
            pltpu.SemaphoreType.DMA,
            pltpu.SemaphoreType.DMA,
            pltpu.SemaphoreType.DMA,
            pltpu.SemaphoreType.DMA,
        ],
        compiler_params=pltpu.CompilerParams(needs_layout_passes=False),
    )(xt_flat, src, dst)


def kernel(x, edge_index):
    xt = x.T.reshape(-1)
    pad = EPAD - N_EDGES
    src = jnp.concatenate([edge_index[0], jnp.zeros((pad,), jnp.int32)])
    dst = jnp.concatenate(
        [edge_index[1], jnp.full((pad,), N_NODES, jnp.int32)])
    out_t = _wl_sc(xt, src, dst)
    return out_t.reshape((NUM_ITERS + 1) * D_FEAT, N_NODES).T
